# baseline (speedup 1.0000x reference)
import jax
import jax.numpy as jnp
from jax import lax
from jax.experimental import pallas as pl
from jax.experimental.pallas import tpu as pltpu
from jax.experimental.pallas import tpu_sc as plsc

_NC = 2
_NS = 16
_NW = _NC * _NS
_K = 4
_NBUF = 2


def _gather_body(table_hbm, idx_hbm, out_hbm, idx_v, rows0, rows1, sem0, sem1):
    wid = lax.axis_index("s") * _NC + lax.axis_index("c")
    nchunk = idx_hbm.shape[1]
    pltpu.sync_copy(idx_hbm.at[wid], idx_v)
    rows = (rows0, rows1)
    sems = (sem0, sem1)
    for b in range(_NBUF):
        pltpu.async_copy(table_hbm.at[idx_v.at[b]], rows[b], sems[b])
    out_base = wid * nchunk

    @pl.loop(0, nchunk, step=_NBUF)
    def _(p):
        for b in range(_NBUF):
            i = p + b
            pltpu.make_async_copy(table_hbm.at[idx_v.at[i]], rows[b], sems[b]).wait()
            pltpu.sync_copy(rows[b], out_hbm.at[out_base + i])
            nxt = i + _NBUF

            @pl.when(nxt < nchunk)
            def _():
                pltpu.async_copy(table_hbm.at[idx_v.at[nxt]], rows[b], sems[b])


def kernel(idx, table):
    b, s = idx.shape
    vocab, d = table.shape
    n_tok = b * s
    nchunk = n_tok // (_NW * _K)
    idx3 = idx.reshape(_NW, nchunk, _K).astype(jnp.int32)
    mesh = plsc.VectorSubcoreMesh(core_axis_name="c", subcore_axis_name="s")
    run = pl.kernel(
        _gather_body,
        out_type=jax.ShapeDtypeStruct((n_tok // _K, _K, d), jnp.float32),
        mesh=mesh,
        scratch_types=[
            pltpu.VMEM((nchunk, _K), jnp.int32),
            pltpu.VMEM((_K, d), jnp.float32),
            pltpu.VMEM((_K, d), jnp.float32),
            pltpu.SemaphoreType.DMA,
            pltpu.SemaphoreType.DMA,
        ],
    )
    out = run(table, idx3)
    return out.reshape(b, s, d)


# speedup vs baseline: 2.4012x; 2.4012x over previous
import jax
import jax.numpy as jnp
from jax import lax
from jax.experimental import pallas as pl
from jax.experimental.pallas import tpu as pltpu
from jax.experimental.pallas import tpu_sc as plsc

_NC = 2
_NS = 16
_NW = _NC * _NS
_K = 4
_NBUF = 2


def _gather_body(table_hbm, idx_hbm, out_hbm, idx_v, rows0, rows1, sem0, sem1):
    wid = lax.axis_index("s") * _NC + lax.axis_index("c")
    nchunk = idx_hbm.shape[1]
    n_seq = out_hbm.shape[1]
    w_per_b = n_seq // (nchunk * _K)
    pltpu.sync_copy(idx_hbm.at[wid], idx_v)
    rows = (rows0, rows1)
    sems = (sem0, sem1)
    for b in range(_NBUF):
        pltpu.async_copy(table_hbm.at[idx_v.at[b]], rows[b], sems[b])
    bb = wid // w_per_b
    s0 = (wid % w_per_b) * (nchunk * _K)

    @pl.loop(0, nchunk, step=_NBUF)
    def _(p):
        for b in range(_NBUF):
            i = p + b
            pltpu.make_async_copy(table_hbm.at[idx_v.at[i]], rows[b], sems[b]).wait()
            pltpu.sync_copy(rows[b], out_hbm.at[bb, pl.ds(s0 + i * _K, _K)])
            nxt = i + _NBUF

            @pl.when(nxt < nchunk)
            def _():
                pltpu.async_copy(table_hbm.at[idx_v.at[nxt]], rows[b], sems[b])


def kernel(idx, table):
    b, s = idx.shape
    vocab, d = table.shape
    n_tok = b * s
    nchunk = n_tok // (_NW * _K)
    idx3 = idx.reshape(_NW, nchunk, _K).astype(jnp.int32)
    mesh = plsc.VectorSubcoreMesh(core_axis_name="c", subcore_axis_name="s")
    run = pl.kernel(
        _gather_body,
        out_type=jax.ShapeDtypeStruct((b, s, d), jnp.float32),
        mesh=mesh,
        scratch_types=[
            pltpu.VMEM((nchunk, _K), jnp.int32),
            pltpu.VMEM((_K, d), jnp.float32),
            pltpu.VMEM((_K, d), jnp.float32),
            pltpu.SemaphoreType.DMA,
            pltpu.SemaphoreType.DMA,
        ],
    )
    return run(table, idx3)
